# Initial kernel scaffold; baseline (speedup 1.0000x reference)
import jax
import jax.numpy as jnp
from jax.experimental import pallas as pl


def kernel(x, context_features, class_features):
    raise NotImplementedError("write your pallas kernel here")


# speedup vs baseline: 1.8771x; 1.8771x over previous
import jax
import jax.numpy as jnp
from jax import lax
from jax.experimental import pallas as pl
from jax.experimental.pallas import tpu as pltpu

_BLK = 512


def _vfg_kernel(cf_ref, x_ref, out_ref, n_ref, m_ref):
    C, D = cf_ref.shape
    K = C // _BLK
    step = pl.program_id(0)

    @pl.when(step == 0)
    def _():
        cf = cf_ref[...]
        nrm = jnp.sqrt(jnp.sum(cf * cf, axis=1, keepdims=True))
        n_ref[...] = cf / jnp.maximum(nrm, 1e-12)
        m_ref[0, 0] = -jnp.inf

    @pl.when(step < K)
    def _():
        i = step
        nb = n_ref[pl.ds(i * _BLK, _BLK), :]
        s = lax.dot_general(nb, n_ref[...], (((1,), (1,)), ((), ())),
                            preferred_element_type=jnp.float32)
        rows = lax.broadcasted_iota(jnp.int32, (_BLK, C), 0) + i * _BLK
        cols = lax.broadcasted_iota(jnp.int32, (_BLK, C), 1)
        s = jnp.where(rows == cols, -jnp.inf, s)
        m_ref[0, 0] = jnp.maximum(m_ref[0, 0], jnp.max(s))

    @pl.when(step == K)
    def _():
        n = n_ref[...]
        xv = x_ref[...]
        g = lax.dot_general(n, xv, (((0,), (0,)), ((), ())),
                            preferred_element_type=jnp.float32)
        d = jnp.sum(n * n, axis=1, keepdims=True)
        y = jnp.dot(n, g, preferred_element_type=jnp.float32) - d * xv
        m = m_ref[0, 0]
        scale = jnp.where(m > 0, 1.0 / jnp.where(m > 0, m, 1.0), 0.1)
        out_ref[...] = y * scale


def kernel(x, context_features, class_features):
    B, C, D = x.shape
    x2 = x.reshape(C, D)
    K = C // _BLK
    out = pl.pallas_call(
        _vfg_kernel,
        grid=(K + 1,),
        in_specs=[
            pl.BlockSpec((C, D), lambda i: (0, 0)),
            pl.BlockSpec((C, D), lambda i: (0, 0)),
        ],
        out_specs=pl.BlockSpec((C, D), lambda i: (0, 0)),
        out_shape=jax.ShapeDtypeStruct((C, D), jnp.float32),
        scratch_shapes=[
            pltpu.VMEM((C, D), jnp.float32),
            pltpu.SMEM((1, 1), jnp.float32),
        ],
        compiler_params=pltpu.CompilerParams(
            dimension_semantics=("arbitrary",),
        ),
    )(context_features, x2)
    return out.reshape(B, C, D)


# speedup vs baseline: 2.2163x; 1.1808x over previous
import jax
import jax.numpy as jnp
from jax import lax
from jax.experimental import pallas as pl
from jax.experimental.pallas import tpu as pltpu

_BLK = 512


def _vfg_kernel(cf_ref, x_ref, out_ref, n_ref, n16_ref, m_ref):
    C, D = cf_ref.shape
    K = C // _BLK
    step = pl.program_id(0)

    @pl.when(step == 0)
    def _():
        cf = cf_ref[...]
        nrm = jnp.sqrt(jnp.sum(cf * cf, axis=1, keepdims=True))
        n = cf / jnp.maximum(nrm, 1e-12)
        n_ref[...] = n
        n16_ref[...] = n.astype(jnp.bfloat16)
        m_ref[0, 0] = -jnp.inf

    @pl.when(step < K)
    def _():
        i = step
        nb = n16_ref[pl.ds(i * _BLK, _BLK), :]
        rows = lax.broadcasted_iota(jnp.int32, (_BLK, _BLK), 0)
        cols = lax.broadcasted_iota(jnp.int32, (_BLK, _BLK), 1)
        diag = rows == cols

        def body(j, acc):
            njb = n16_ref[pl.ds(j * _BLK, _BLK), :]
            s = lax.dot_general(nb, njb, (((1,), (1,)), ((), ())),
                                preferred_element_type=jnp.float32)
            s = jnp.where((j == i) & diag, -jnp.inf, s)
            return jnp.maximum(acc, s)

        acc = lax.fori_loop(i, K, body,
                            jnp.full((_BLK, _BLK), -jnp.inf, jnp.float32))
        m_ref[0, 0] = jnp.maximum(m_ref[0, 0], jnp.max(acc))

    @pl.when(step == K)
    def _():
        n = n_ref[...]
        xv = x_ref[...]
        g = lax.dot_general(n, xv, (((0,), (0,)), ((), ())),
                            preferred_element_type=jnp.float32)
        d = jnp.sum(n * n, axis=1, keepdims=True)
        y = jnp.dot(n, g, preferred_element_type=jnp.float32) - d * xv
        m = m_ref[0, 0]
        scale = jnp.where(m > 0, 1.0 / jnp.where(m > 0, m, 1.0), 0.1)
        out_ref[...] = y * scale


def kernel(x, context_features, class_features):
    B, C, D = x.shape
    x2 = x.reshape(C, D)
    K = C // _BLK
    out = pl.pallas_call(
        _vfg_kernel,
        grid=(K + 1,),
        in_specs=[
            pl.BlockSpec((C, D), lambda i: (0, 0)),
            pl.BlockSpec((C, D), lambda i: (0, 0)),
        ],
        out_specs=pl.BlockSpec((C, D), lambda i: (0, 0)),
        out_shape=jax.ShapeDtypeStruct((C, D), jnp.float32),
        scratch_shapes=[
            pltpu.VMEM((C, D), jnp.float32),
            pltpu.VMEM((C, D), jnp.bfloat16),
            pltpu.SMEM((1, 1), jnp.float32),
        ],
        compiler_params=pltpu.CompilerParams(
            dimension_semantics=("arbitrary",),
        ),
    )(context_features, x2)
    return out.reshape(B, C, D)


# speedup vs baseline: 2.2737x; 1.0259x over previous
import jax
import jax.numpy as jnp
from jax import lax
from jax.experimental import pallas as pl
from jax.experimental.pallas import tpu as pltpu

_BLK = 512


def _vfg_kernel(cf_ref, x_ref, out_ref, n_ref, n16_ref, m_ref):
    C, D = cf_ref.shape
    K = C // _BLK
    step = pl.program_id(0)

    @pl.when(step == 0)
    def _():
        cf = cf_ref[...]
        nrm = jnp.sqrt(jnp.sum(cf * cf, axis=1, keepdims=True))
        n = cf / jnp.maximum(nrm, 1e-12)
        n_ref[...] = n
        n16_ref[...] = n.astype(jnp.bfloat16)
        m_ref[0, 0] = -jnp.inf

    @pl.when(step < K)
    def _():
        i = step
        nb = n16_ref[pl.ds(i * _BLK, _BLK), :]
        rows = lax.broadcasted_iota(jnp.int32, (_BLK, _BLK), 0)
        cols = lax.broadcasted_iota(jnp.int32, (_BLK, _BLK), 1)
        sd = lax.dot_general(nb, nb, (((1,), (1,)), ((), ())),
                             preferred_element_type=jnp.float32)
        acc0 = jnp.where(rows == cols, -jnp.inf, sd)

        def body(j, acc):
            njb = n16_ref[pl.ds(j * _BLK, _BLK), :]
            s = lax.dot_general(nb, njb, (((1,), (1,)), ((), ())),
                                preferred_element_type=jnp.float32)
            return jnp.maximum(acc, s)

        acc = lax.fori_loop(i + 1, K, body, acc0)
        m_ref[0, 0] = jnp.maximum(m_ref[0, 0], jnp.max(acc))

    @pl.when(step == K)
    def _():
        n = n_ref[...]
        xv = x_ref[...]
        g = lax.dot_general(n, xv, (((0,), (0,)), ((), ())),
                            preferred_element_type=jnp.float32)
        d = jnp.sum(n * n, axis=1, keepdims=True)
        y = jnp.dot(n, g, preferred_element_type=jnp.float32) - d * xv
        m = m_ref[0, 0]
        scale = jnp.where(m > 0, 1.0 / jnp.where(m > 0, m, 1.0), 0.1)
        out_ref[...] = y * scale


def kernel(x, context_features, class_features):
    B, C, D = x.shape
    x2 = x.reshape(C, D)
    K = C // _BLK
    out = pl.pallas_call(
        _vfg_kernel,
        grid=(K + 1,),
        in_specs=[
            pl.BlockSpec((C, D), lambda i: (0, 0)),
            pl.BlockSpec((C, D), lambda i: (0, 0)),
        ],
        out_specs=pl.BlockSpec((C, D), lambda i: (0, 0)),
        out_shape=jax.ShapeDtypeStruct((C, D), jnp.float32),
        scratch_shapes=[
            pltpu.VMEM((C, D), jnp.float32),
            pltpu.VMEM((C, D), jnp.bfloat16),
            pltpu.SMEM((1, 1), jnp.float32),
        ],
        compiler_params=pltpu.CompilerParams(
            dimension_semantics=("arbitrary",),
        ),
    )(context_features, x2)
    return out.reshape(B, C, D)


# speedup vs baseline: 2.9135x; 1.2814x over previous
import jax
import jax.numpy as jnp
from jax import lax
from jax.experimental import pallas as pl
from jax.experimental.pallas import tpu as pltpu

_BLK = 512


def _vfg_kernel(cf_ref, x_ref, out_ref, n_ref, n16_ref, m_ref):
    C, D = cf_ref.shape
    K = C // _BLK
    step = pl.program_id(0)

    @pl.when(step == 0)
    def _():
        cf = cf_ref[...]
        nrm = jnp.sqrt(jnp.sum(cf * cf, axis=1, keepdims=True))
        n = cf / jnp.maximum(nrm, 1e-12)
        n_ref[...] = n
        n16_ref[...] = n.astype(jnp.bfloat16)
        m_ref[0, 0] = -jnp.inf

    @pl.when(step < K)
    def _():
        i = step
        nb = n16_ref[pl.ds(i * _BLK, _BLK), :]
        rows = lax.broadcasted_iota(jnp.int32, (_BLK, _BLK), 0)
        cols = lax.broadcasted_iota(jnp.int32, (_BLK, _BLK), 1)
        sd = lax.dot_general(nb, nb, (((1,), (1,)), ((), ())),
                             preferred_element_type=jnp.float32)
        r0 = jnp.max(jnp.where(rows == cols, -jnp.inf, sd), axis=0)

        def body(j, r):
            njb = n16_ref[pl.ds(j * _BLK, _BLK), :]
            s = lax.dot_general(nb, njb, (((1,), (1,)), ((), ())),
                                preferred_element_type=jnp.float32)
            return jnp.maximum(r, jnp.max(s, axis=0))

        r = lax.fori_loop(i + 1, K, body, r0)
        m_ref[0, 0] = jnp.maximum(m_ref[0, 0], jnp.max(r))

    @pl.when(step == K)
    def _():
        n = n_ref[...]
        xv = x_ref[...]
        g = lax.dot_general(n, xv, (((0,), (0,)), ((), ())),
                            preferred_element_type=jnp.float32)
        d = jnp.sum(n * n, axis=1, keepdims=True)
        y = jnp.dot(n, g, preferred_element_type=jnp.float32) - d * xv
        m = m_ref[0, 0]
        scale = jnp.where(m > 0, 1.0 / jnp.where(m > 0, m, 1.0), 0.1)
        out_ref[...] = y * scale


def kernel(x, context_features, class_features):
    B, C, D = x.shape
    x2 = x.reshape(C, D)
    K = C // _BLK
    out = pl.pallas_call(
        _vfg_kernel,
        grid=(K + 1,),
        in_specs=[
            pl.BlockSpec((C, D), lambda i: (0, 0)),
            pl.BlockSpec((C, D), lambda i: (0, 0)),
        ],
        out_specs=pl.BlockSpec((C, D), lambda i: (0, 0)),
        out_shape=jax.ShapeDtypeStruct((C, D), jnp.float32),
        scratch_shapes=[
            pltpu.VMEM((C, D), jnp.float32),
            pltpu.VMEM((C, D), jnp.bfloat16),
            pltpu.SMEM((1, 1), jnp.float32),
        ],
        compiler_params=pltpu.CompilerParams(
            dimension_semantics=("arbitrary",),
        ),
    )(context_features, x2)
    return out.reshape(B, C, D)


# speedup vs baseline: 3.7289x; 1.2799x over previous
import jax
import jax.numpy as jnp
from jax import lax
from jax.experimental import pallas as pl
from jax.experimental.pallas import tpu as pltpu

_BLK = 512
_CHUNK = 2048


def _vfg_kernel(cf_ref, x_ref, out_ref, n_ref, n16_ref, m_ref):
    C, D = cf_ref.shape
    K = C // _BLK
    step = pl.program_id(0)

    @pl.when(step == 0)
    def _():
        cf = cf_ref[...]
        nrm = jnp.sqrt(jnp.sum(cf * cf, axis=1, keepdims=True))
        n = cf / jnp.maximum(nrm, 1e-12)
        n_ref[...] = n
        n16_ref[...] = n.astype(jnp.bfloat16)
        m_ref[0, 0] = -jnp.inf

    @pl.when(step < K)
    def _():
        i = step
        nb = n16_ref[pl.ds(i * _BLK, _BLK), :]
        rows = lax.broadcasted_iota(jnp.int32, (_BLK, _CHUNK), 0) + i * _BLK
        cols = lax.broadcasted_iota(jnp.int32, (_BLK, _CHUNK), 1)

        c0 = (i * _BLK) // _CHUNK
        nd = n16_ref[pl.ds(c0 * _CHUNK, _CHUNK), :]
        sd = lax.dot_general(nb, nd, (((1,), (1,)), ((), ())),
                             preferred_element_type=jnp.float32)
        sd = jnp.where(rows == cols + c0 * _CHUNK, -jnp.inf, sd)
        r0 = jnp.max(sd, axis=0)

        def body(j, r):
            njb = n16_ref[pl.ds(j * _CHUNK, _CHUNK), :]
            s = lax.dot_general(nb, njb, (((1,), (1,)), ((), ())),
                                preferred_element_type=jnp.float32)
            return jnp.maximum(r, jnp.max(s, axis=0))

        r = lax.fori_loop(c0 + 1, C // _CHUNK, body, r0)
        m_ref[0, 0] = jnp.maximum(m_ref[0, 0], jnp.max(r))

    @pl.when(step == K)
    def _():
        n = n_ref[...]
        xv = x_ref[...]
        g = lax.dot_general(n, xv, (((0,), (0,)), ((), ())),
                            preferred_element_type=jnp.float32)
        d = jnp.sum(n * n, axis=1, keepdims=True)
        y = jnp.dot(n, g, preferred_element_type=jnp.float32) - d * xv
        m = m_ref[0, 0]
        scale = jnp.where(m > 0, 1.0 / jnp.where(m > 0, m, 1.0), 0.1)
        out_ref[...] = y * scale


def kernel(x, context_features, class_features):
    B, C, D = x.shape
    x2 = x.reshape(C, D)
    K = C // _BLK
    out = pl.pallas_call(
        _vfg_kernel,
        grid=(K + 1,),
        in_specs=[
            pl.BlockSpec((C, D), lambda i: (0, 0)),
            pl.BlockSpec((C, D), lambda i: (0, 0)),
        ],
        out_specs=pl.BlockSpec((C, D), lambda i: (0, 0)),
        out_shape=jax.ShapeDtypeStruct((C, D), jnp.float32),
        scratch_shapes=[
            pltpu.VMEM((C, D), jnp.float32),
            pltpu.VMEM((C, D), jnp.bfloat16),
            pltpu.SMEM((1, 1), jnp.float32),
        ],
        compiler_params=pltpu.CompilerParams(
            dimension_semantics=("arbitrary",),
        ),
    )(context_features, x2)
    return out.reshape(B, C, D)


# speedup vs baseline: 4.1613x; 1.1160x over previous
import jax
import jax.numpy as jnp
from jax import lax
from jax.experimental import pallas as pl
from jax.experimental.pallas import tpu as pltpu

_BLK = 1024
_CHUNK = 2048


def _vfg_kernel(cf_ref, x_ref, out_ref, n_ref, n16_ref, m_ref):
    C, D = cf_ref.shape
    K = C // _BLK
    step = pl.program_id(0)

    @pl.when(step == 0)
    def _():
        cf = cf_ref[...]
        nrm = jnp.sqrt(jnp.sum(cf * cf, axis=1, keepdims=True))
        n = cf / jnp.maximum(nrm, 1e-12)
        n_ref[...] = n
        n16_ref[...] = n.astype(jnp.bfloat16)
        m_ref[0, 0] = -jnp.inf

    @pl.when(step < K)
    def _():
        i = step
        nb = n16_ref[pl.ds(i * _BLK, _BLK), :]
        rows = lax.broadcasted_iota(jnp.int32, (_BLK, _CHUNK), 0) + i * _BLK
        cols = lax.broadcasted_iota(jnp.int32, (_BLK, _CHUNK), 1)

        c0 = (i * _BLK) // _CHUNK
        nd = n16_ref[pl.ds(c0 * _CHUNK, _CHUNK), :]
        sd = lax.dot_general(nb, nd, (((1,), (1,)), ((), ())),
                             preferred_element_type=jnp.float32)
        sd = jnp.where(rows == cols + c0 * _CHUNK, -jnp.inf, sd)
        r0 = jnp.max(sd, axis=0)

        def body(j, r):
            njb = n16_ref[pl.ds(j * _CHUNK, _CHUNK), :]
            s = lax.dot_general(nb, njb, (((1,), (1,)), ((), ())),
                                preferred_element_type=jnp.float32)
            return jnp.maximum(r, jnp.max(s, axis=0))

        r = lax.fori_loop(c0 + 1, C // _CHUNK, body, r0)
        m_ref[0, 0] = jnp.maximum(m_ref[0, 0], jnp.max(r))

    @pl.when(step == K)
    def _():
        n = n_ref[...]
        xv = x_ref[...]
        g = lax.dot_general(n, xv, (((0,), (0,)), ((), ())),
                            preferred_element_type=jnp.float32)
        d = jnp.sum(n * n, axis=1, keepdims=True)
        y = jnp.dot(n, g, preferred_element_type=jnp.float32) - d * xv
        m = m_ref[0, 0]
        scale = jnp.where(m > 0, 1.0 / jnp.where(m > 0, m, 1.0), 0.1)
        out_ref[...] = y * scale


def kernel(x, context_features, class_features):
    B, C, D = x.shape
    x2 = x.reshape(C, D)
    K = C // _BLK
    out = pl.pallas_call(
        _vfg_kernel,
        grid=(K + 1,),
        in_specs=[
            pl.BlockSpec((C, D), lambda i: (0, 0)),
            pl.BlockSpec((C, D), lambda i: (0, 0)),
        ],
        out_specs=pl.BlockSpec((C, D), lambda i: (0, 0)),
        out_shape=jax.ShapeDtypeStruct((C, D), jnp.float32),
        scratch_shapes=[
            pltpu.VMEM((C, D), jnp.float32),
            pltpu.VMEM((C, D), jnp.bfloat16),
            pltpu.SMEM((1, 1), jnp.float32),
        ],
        compiler_params=pltpu.CompilerParams(
            dimension_semantics=("arbitrary",),
        ),
    )(context_features, x2)
    return out.reshape(B, C, D)


# speedup vs baseline: 4.3982x; 1.0569x over previous
import jax
import jax.numpy as jnp
from jax import lax
from jax.experimental import pallas as pl
from jax.experimental.pallas import tpu as pltpu

_BLK = 2048
_CHUNK = 2048


def _vfg_kernel(cf_ref, x_ref, out_ref, n_ref, n16_ref, m_ref):
    C, D = cf_ref.shape
    K = C // _BLK
    step = pl.program_id(0)

    @pl.when(step == 0)
    def _():
        cf = cf_ref[...]
        nrm = jnp.sqrt(jnp.sum(cf * cf, axis=1, keepdims=True))
        n = cf / jnp.maximum(nrm, 1e-12)
        n_ref[...] = n
        n16_ref[...] = n.astype(jnp.bfloat16)
        m_ref[0, 0] = -jnp.inf

    @pl.when(step < K)
    def _():
        i = step
        nb = n16_ref[pl.ds(i * _BLK, _BLK), :]
        rows = lax.broadcasted_iota(jnp.int32, (_BLK, _CHUNK), 0) + i * _BLK
        cols = lax.broadcasted_iota(jnp.int32, (_BLK, _CHUNK), 1)

        c0 = (i * _BLK) // _CHUNK
        nd = n16_ref[pl.ds(c0 * _CHUNK, _CHUNK), :]
        sd = lax.dot_general(nb, nd, (((1,), (1,)), ((), ())),
                             preferred_element_type=jnp.float32)
        sd = jnp.where(rows == cols + c0 * _CHUNK, -jnp.inf, sd)
        r0 = jnp.max(sd, axis=0)

        def body(j, r):
            njb = n16_ref[pl.ds(j * _CHUNK, _CHUNK), :]
            s = lax.dot_general(nb, njb, (((1,), (1,)), ((), ())),
                                preferred_element_type=jnp.float32)
            return jnp.maximum(r, jnp.max(s, axis=0))

        r = lax.fori_loop(c0 + 1, C // _CHUNK, body, r0)
        m_ref[0, 0] = jnp.maximum(m_ref[0, 0], jnp.max(r))

    @pl.when(step == K)
    def _():
        n = n_ref[...]
        xv = x_ref[...]
        g = lax.dot_general(n, xv, (((0,), (0,)), ((), ())),
                            preferred_element_type=jnp.float32)
        d = jnp.sum(n * n, axis=1, keepdims=True)
        y = jnp.dot(n, g, preferred_element_type=jnp.float32) - d * xv
        m = m_ref[0, 0]
        scale = jnp.where(m > 0, 1.0 / jnp.where(m > 0, m, 1.0), 0.1)
        out_ref[...] = y * scale


def kernel(x, context_features, class_features):
    B, C, D = x.shape
    x2 = x.reshape(C, D)
    K = C // _BLK
    out = pl.pallas_call(
        _vfg_kernel,
        grid=(K + 1,),
        in_specs=[
            pl.BlockSpec((C, D), lambda i: (0, 0)),
            pl.BlockSpec((C, D), lambda i: (0, 0)),
        ],
        out_specs=pl.BlockSpec((C, D), lambda i: (0, 0)),
        out_shape=jax.ShapeDtypeStruct((C, D), jnp.float32),
        scratch_shapes=[
            pltpu.VMEM((C, D), jnp.float32),
            pltpu.VMEM((C, D), jnp.bfloat16),
            pltpu.SMEM((1, 1), jnp.float32),
        ],
        compiler_params=pltpu.CompilerParams(
            dimension_semantics=("arbitrary",),
        ),
    )(context_features, x2)
    return out.reshape(B, C, D)


# speedup vs baseline: 4.6993x; 1.0685x over previous
import jax
import jax.numpy as jnp
from jax import lax
from jax.experimental import pallas as pl
from jax.experimental.pallas import tpu as pltpu

_BLK = 2048
_CHUNK = 2048
_SUB = 512


def _vfg_kernel(cf_ref, x_ref, out_ref, n_ref, n16_ref, m_ref):
    C, D = cf_ref.shape
    K = C // _BLK
    step = pl.program_id(0)

    @pl.when(step == 0)
    def _():
        cf = cf_ref[...]
        nrm2 = jnp.sum(cf * cf, axis=1, keepdims=True)
        n = cf * lax.rsqrt(jnp.maximum(nrm2, 1e-24))
        n_ref[...] = n
        n16_ref[...] = n.astype(jnp.bfloat16)
        m_ref[0, 0] = -jnp.inf

    @pl.when(step < K)
    def _():
        i = step
        base = i * _BLK
        nb = n16_ref[pl.ds(base, _BLK), :]

        mloc = jnp.float32(-jnp.inf)
        for a in range(_BLK // _SUB):
            width = _BLK - a * _SUB
            rb = n16_ref[pl.ds(base + a * _SUB, _SUB), :]
            cb = n16_ref[pl.ds(base + a * _SUB, width), :]
            s = lax.dot_general(rb, cb, (((1,), (1,)), ((), ())),
                                preferred_element_type=jnp.float32)
            rr = lax.broadcasted_iota(jnp.int32, (_SUB, width), 0)
            cc = lax.broadcasted_iota(jnp.int32, (_SUB, width), 1)
            s = jnp.where(rr == cc, -jnp.inf, s)
            mloc = jnp.maximum(mloc, jnp.max(s))

        def body(j, r):
            njb = n16_ref[pl.ds(j * _CHUNK, _CHUNK), :]
            s = lax.dot_general(nb, njb, (((1,), (1,)), ((), ())),
                                preferred_element_type=jnp.float32)
            return jnp.maximum(r, jnp.max(s, axis=0))

        r = lax.fori_loop(i + 1, C // _CHUNK, body,
                          jnp.full((_CHUNK,), -jnp.inf, jnp.float32))
        m_ref[0, 0] = jnp.maximum(m_ref[0, 0],
                                  jnp.maximum(mloc, jnp.max(r)))

    @pl.when(step == K)
    def _():
        n = n_ref[...]
        xv = x_ref[...]
        g = lax.dot_general(n, xv, (((0,), (0,)), ((), ())),
                            preferred_element_type=jnp.float32)
        d = jnp.sum(n * n, axis=1, keepdims=True)
        y = jnp.dot(n, g, preferred_element_type=jnp.float32) - d * xv
        m = m_ref[0, 0]
        scale = jnp.where(m > 0, 1.0 / jnp.where(m > 0, m, 1.0), 0.1)
        out_ref[...] = y * scale


def kernel(x, context_features, class_features):
    B, C, D = x.shape
    x2 = x.reshape(C, D)
    K = C // _BLK
    out = pl.pallas_call(
        _vfg_kernel,
        grid=(K + 1,),
        in_specs=[
            pl.BlockSpec((C, D), lambda i: (0, 0)),
            pl.BlockSpec((C, D), lambda i: (0, 0)),
        ],
        out_specs=pl.BlockSpec((C, D), lambda i: (0, 0)),
        out_shape=jax.ShapeDtypeStruct((C, D), jnp.float32),
        scratch_shapes=[
            pltpu.VMEM((C, D), jnp.float32),
            pltpu.VMEM((C, D), jnp.bfloat16),
            pltpu.SMEM((1, 1), jnp.float32),
        ],
        compiler_params=pltpu.CompilerParams(
            dimension_semantics=("arbitrary",),
        ),
    )(context_features, x2)
    return out.reshape(B, C, D)
